# Initial kernel scaffold; baseline (speedup 1.0000x reference)
import jax
import jax.numpy as jnp
from jax.experimental import pallas as pl


def kernel(node_features, edge_indices, edge_features, xbatch, params):
    raise NotImplementedError("write your pallas kernel here")


# speedup vs baseline: 5.8037x; 5.8037x over previous
import functools

import jax
import jax.numpy as jnp
import numpy as np
from jax import lax
from jax.experimental import pallas as pl
from jax.experimental.pallas import tpu as pltpu
from jax.experimental.pallas import tpu_sc as plsc

N = 10000
E = 160000
D = 16
PACK = 8
ROWS = E // PACK
NUM_MP = 3

NC = 2
NS = 16
NW = NC * NS
CHUNK = E // NW
PRC = CHUNK // PACK
BAT = 40
NBAT = CHUNK // BAT
STRIPE = N // NS

RB = 2000
GRID = ROWS // RB
MBP = 2000
MGRID = ROWS // MBP

F32 = jnp.float32

_m = np.arange(128)
P128 = ((_m[:, None] % D) == (_m[None, :] % D)).astype(np.float32) / float(E)
_I8 = np.eye(PACK, dtype=np.float32)
BIS = np.zeros((16, 128, 128), np.float32)
for _i in range(16):
    _O = np.zeros((16, 16), np.float32)
    _O[_i, :] = 1.0
    BIS[_i] = np.kron(_I8, _O)


def _elu(x):
    return jnp.where(x > 0, x, jnp.exp(x) - 1.0)


def _dot(a, b):
    return jnp.dot(a, b, preferred_element_type=F32)


def _dot_hi(a, b):
    return jnp.dot(a, b, preferred_element_type=F32,
                   precision=lax.Precision.HIGHEST)


def _fold(st, P):
    mu = _dot_hi(st[0:1, :], P)
    ex2 = _dot_hi(st[1:2, :], P)
    rs = lax.rsqrt(jnp.maximum(ex2 - mu * mu, 0.0) + 1e-5)
    return mu, rs


def _bn_pk(x, st, P, g, b):
    mu, rs = _fold(st, P)
    return (x - mu) * rs * g + b


def _acc_stats(st_o, y):
    @pl.when(pl.program_id(0) == 0)
    def _():
        st_o[...] = jnp.zeros_like(st_o)

    st_o[0:1, :] += jnp.sum(y, axis=0, keepdims=True)
    st_o[1:2, :] += jnp.sum(y * y, axis=0, keepdims=True)


def _stats1_body(x, st_o):
    _acc_stats(st_o, x[...])


def _stats2_body(a, b, sa_o, sb_o):
    _acc_stats(sa_o, a[...])
    _acc_stats(sb_o, b[...])


def _stage_lin_body(x, st, P, g, b, W, bias, out, st_o):
    y = _elu(_dot(_bn_pk(x[...], st[...], P[...], g[...], b[...]), W[...])
             + bias[...])
    out[...] = y
    _acc_stats(st_o, y)


def _stage_el1_body(xs, xd, e, sxs, sxd, se, P, ga, ba, gb, bb, gc, bc,
                    Wa, Wb, Wc, bias, out, st_o):
    Pv = P[...]
    acc = _dot(_bn_pk(xs[...], sxs[...], Pv, ga[...], ba[...]), Wa[...])
    acc += _dot(_bn_pk(xd[...], sxd[...], Pv, gb[...], bb[...]), Wb[...])
    acc += _dot(_bn_pk(e[...], se[...], Pv, gc[...], bc[...]), Wc[...])
    y = _elu(acc + bias[...])
    out[...] = y
    _acc_stats(st_o, y)


def _stage_el3_body(x, st, P, g, b, W, bias, out, st_o):
    y = _dot(_bn_pk(x[...], st[...], P[...], g[...], b[...]), W[...]) + bias[...]
    out[...] = y
    _acc_stats(st_o, y)


def _stage_el3_last_body(x, st, P, g, b, W, bias, PWp, pb, out, pred_o):
    y = _dot(_bn_pk(x[...], st[...], P[...], g[...], b[...]), W[...]) + bias[...]
    out[...] = y
    pred_o[...] = _dot(y, PWp[...]) + pb[...]


def _msg_body(xj, h2, st, P, g, b, W2s, b2s, Bis, out):
    h2b = _bn_pk(h2[...], st[...], P[...], g[...], b[...])
    xjv = xj[...]
    acc = None
    for i in range(D):
        wf = _dot(h2b, W2s[i]) + b2s[i:i + 1, :]
        wfb = wf.astype(jnp.bfloat16).astype(F32)
        xb_i = _dot(xjv, Bis[i])
        term = xb_i * wfb
        acc = term if acc is None else acc + term
    out[...] = acc


def _node_prep_body(x, g, b, root, cb, xb_o, xr_o):
    xv = x[...]
    mu = jnp.mean(xv, axis=0, keepdims=True)
    c = xv - mu
    var = jnp.mean(c * c, axis=0, keepdims=True)
    xb = c * lax.rsqrt(var + 1e-5) * g[...] + b[...]
    xb_o[...] = xb
    xr_o[...] = _dot(xb, root[...]) + cb[...]


def _node_update_mid_body(agg2, xr, g, b, root, cb, xn_o, xb_o, xr_o):
    xn = _elu(agg2[0] + agg2[1] + xr[...])
    xn_o[...] = xn
    mu = jnp.mean(xn, axis=0, keepdims=True)
    c = xn - mu
    var = jnp.mean(c * c, axis=0, keepdims=True)
    xb = c * lax.rsqrt(var + 1e-5) * g[...] + b[...]
    xb_o[...] = xb
    xr_o[...] = _dot(xb, root[...]) + cb[...]


def _node_update_last_body(agg2, xr, npw, npb, xn_o, xp_o):
    xn = _elu(agg2[0] + agg2[1] + xr[...])
    xn_o[...] = xn
    xp_o[...] = _dot(xn, npw[...]) + npb[...]


def _bsp(g):
    return (g, 0)


def _csp(g):
    return (0, 0)


def _csp3(g):
    return (0, 0, 0)


_PK = pl.BlockSpec((RB, 128), _bsp)
_ST = pl.BlockSpec((2, 128), _csp)


def _c2(shape):
    return pl.BlockSpec(shape, _csp)


_PKSHAPE = jax.ShapeDtypeStruct((ROWS, 128), F32)
_STSHAPE = jax.ShapeDtypeStruct((2, 128), F32)


def _stats1(x):
    return pl.pallas_call(
        _stats1_body, grid=(GRID,), in_specs=[_PK], out_specs=_ST,
        out_shape=_STSHAPE)(x)


def _stats2(a, b):
    return pl.pallas_call(
        _stats2_body, grid=(GRID,), in_specs=[_PK, _PK],
        out_specs=[_ST, _ST], out_shape=[_STSHAPE, _STSHAPE])(a, b)


def _stage_lin(x, st, P, g, b, W, bias, elu3=False):
    body = _stage_el3_body if elu3 else _stage_lin_body
    return pl.pallas_call(
        body, grid=(GRID,),
        in_specs=[_PK, _ST, _c2((128, 128)), _c2((1, 128)), _c2((1, 128)),
                  _c2((128, 128)), _c2((1, 128))],
        out_specs=[_PK, _ST],
        out_shape=[_PKSHAPE, _STSHAPE])(x, st, P, g, b, W, bias)


def _stage_el1(xs, xd, e, sxs, sxd, se, P, args):
    return pl.pallas_call(
        _stage_el1_body, grid=(GRID,),
        in_specs=[_PK, _PK, _PK, _ST, _ST, _ST, _c2((128, 128))]
        + [_c2((1, 128))] * 6 + [_c2((128, 128))] * 3 + [_c2((1, 128))],
        out_specs=[_PK, _ST],
        out_shape=[_PKSHAPE, _STSHAPE])(xs, xd, e, sxs, sxd, se, P, *args)


def _stage_el3_last(x, st, P, g, b, W, bias, PWp, pb):
    return pl.pallas_call(
        _stage_el3_last_body, grid=(GRID,),
        in_specs=[_PK, _ST, _c2((128, 128)), _c2((1, 128)), _c2((1, 128)),
                  _c2((128, 128)), _c2((1, 128)), _c2((128, 16)),
                  _c2((1, 16))],
        out_specs=[_PK, pl.BlockSpec((RB, 16), _bsp)],
        out_shape=[_PKSHAPE, jax.ShapeDtypeStruct((ROWS, 16), F32)])(
            x, st, P, g, b, W, bias, PWp, pb)


_MPK = pl.BlockSpec((MBP, 128), _bsp)


def _msg(xj, h2, st, P, g, b, W2s, b2s, Bis):
    return pl.pallas_call(
        _msg_body, grid=(MGRID,),
        in_specs=[_MPK, _MPK, pl.BlockSpec((2, 128), _csp),
                  _c2((128, 128)), _c2((1, 128)), _c2((1, 128)),
                  pl.BlockSpec((16, 128, 128), _csp3), _c2((16, 128)),
                  pl.BlockSpec((16, 128, 128), _csp3)],
        out_specs=_MPK,
        out_shape=_PKSHAPE)(xj, h2, st, P, g, b, W2s, b2s, Bis)


@functools.lru_cache(maxsize=1)
def _sc_kernels():
    mesh = plsc.VectorSubcoreMesh(core_axis_name="c", subcore_axis_name="s",
                                  num_cores=NC, num_subcores=NS)
    cparams = pltpu.CompilerParams(use_tc_tiling_on_sc=False)

    def gather_one(table, idx_hbm, out_hbm, wid, idx_v, rows_v, sem):
        pltpu.sync_copy(idx_hbm.at[wid], idx_v)

        def fire(j, _):
            pltpu.async_copy(table.at[idx_v.at[j]],
                             rows_v.at[pl.ds(j * BAT, BAT)], sem)
            return 0

        lax.fori_loop(0, NBAT, fire, 0)
        pltpu.make_async_copy(table.at[pl.ds(0, CHUNK)], rows_v, sem).wait()
        pltpu.sync_copy(rows_v, out_hbm.at[pl.ds(wid * CHUNK, CHUNK)])

    scratch = [pltpu.VMEM((NBAT, BAT), jnp.int32),
               pltpu.VMEM((CHUNK, D), F32),
               pltpu.SemaphoreType.DMA]

    @functools.partial(
        pl.kernel,
        out_type=jax.ShapeDtypeStruct((E, D), F32),
        mesh=mesh, scratch_types=scratch, compiler_params=cparams)
    def gather1(table, rowi, out_r, idx_v, rows_v, sem):
        wid = lax.axis_index("c") * NS + lax.axis_index("s")
        gather_one(table, rowi, out_r, wid, idx_v, rows_v, sem)

    @functools.partial(
        pl.kernel,
        out_type=[jax.ShapeDtypeStruct((E, D), F32),
                  jax.ShapeDtypeStruct((E, D), F32)],
        mesh=mesh, scratch_types=scratch, compiler_params=cparams)
    def gather2(table, rowi, coli, out_r, out_c, idx_v, rows_v, sem):
        wid = lax.axis_index("c") * NS + lax.axis_index("s")
        gather_one(table, rowi, out_r, wid, idx_v, rows_v, sem)
        gather_one(table, coli, out_c, wid, idx_v, rows_v, sem)

    @functools.partial(
        pl.kernel,
        out_type=jax.ShapeDtypeStruct((NC, N, D), F32),
        mesh=mesh,
        scratch_types=[pltpu.VMEM((NBAT, BAT), jnp.int32),
                       pltpu.VMEM((CHUNK, D), F32),
                       pltpu.VMEM_SHARED((N, D), F32),
                       pltpu.SemaphoreType.DMA],
        compiler_params=cparams)
    def scatter(msg16, coli, zeros, out, idx_v, msg_v, shared, sem):
        cid = lax.axis_index("c")
        sid = lax.axis_index("s")
        wid = cid * NS + sid

        pltpu.sync_copy(zeros.at[pl.ds(sid * STRIPE, STRIPE)],
                        shared.at[pl.ds(sid * STRIPE, STRIPE)])
        plsc.subcore_barrier()

        pltpu.sync_copy(coli.at[wid], idx_v)
        pltpu.sync_copy(msg16.at[pl.ds(wid * CHUNK, CHUNK)], msg_v)

        def fire(j, _):
            pltpu.async_copy(msg_v.at[pl.ds(j * BAT, BAT)],
                             shared.at[idx_v.at[j]], sem, add=True)
            return 0

        lax.fori_loop(0, NBAT, fire, 0)
        pltpu.make_async_copy(msg16.at[pl.ds(wid * CHUNK, CHUNK)], msg_v,
                              sem).wait()
        plsc.subcore_barrier()
        pltpu.sync_copy(shared.at[pl.ds(sid * STRIPE, STRIPE)],
                        out.at[cid, pl.ds(sid * STRIPE, STRIPE)])

    return gather1, gather2, scatter


def _sc_gather1(table, rowi):
    return _sc_kernels()[0](table, rowi)


def _sc_gather2(table, rowi, coli):
    return _sc_kernels()[1](table, rowi, coli)


def _sc_scatter(msg_p, coli, zeros):
    return _sc_kernels()[2](msg_p, coli, zeros)


def _prep_layer(p):
    q = {}
    t8 = lambda v: jnp.tile(v, PACK).reshape(1, 128)
    eye8 = jnp.asarray(_I8)
    k8 = lambda W: jnp.kron(eye8, W)
    r2 = lambda v: v.reshape(1, -1)
    q['bn_g'], q['bn_b'] = r2(p['bn_node_g']), r2(p['bn_node_b'])
    q['root'], q['cb'] = p['root'], r2(p['conv_bias'])
    q['em0'] = (t8(p['em_bn0_g']), t8(p['em_bn0_b']), k8(p['em_w0']),
                t8(p['em_b0']))
    q['em1'] = (t8(p['em_bn1_g']), t8(p['em_bn1_b']), k8(p['em_w1']),
                t8(p['em_b1']))
    q['m_g'], q['m_b'] = t8(p['em_bn2_g']), t8(p['em_bn2_b'])
    q['W2s'] = jnp.stack([k8(p['em_w2'][:, i * 16:(i + 1) * 16])
                          for i in range(16)])
    q['b2s'] = jnp.stack([jnp.tile(p['em_b2'][i * 16:(i + 1) * 16], PACK)
                          for i in range(16)])
    g0, b0, w0 = p['el_bn0_g'], p['el_bn0_b'], p['el_w0']
    q['el1'] = (t8(g0[:16]), t8(b0[:16]), t8(g0[16:32]), t8(b0[16:32]),
                t8(g0[32:]), t8(b0[32:]), k8(w0[:16]), k8(w0[16:32]),
                k8(w0[32:]), t8(p['el_b0']))
    q['el2'] = (t8(p['el_bn1_g']), t8(p['el_bn1_b']), k8(p['el_w1']),
                t8(p['el_b1']))
    q['el3'] = (t8(p['el_bn2_g']), t8(p['el_bn2_b']), k8(p['el_w2']),
                t8(p['el_b2']))
    return q


def kernel(node_features, edge_indices, edge_features, xbatch, params):
    x = node_features.reshape(N, D)
    e_p = edge_features.reshape(ROWS, 128)
    rowi = edge_indices[0].reshape(NW, NBAT, BAT)
    coli = edge_indices[1].reshape(NW, NBAT, BAT)
    zeros = jnp.zeros((N, D), F32)
    qs = [_prep_layer(p) for p in params['mp']]
    P = jnp.asarray(P128)
    eye8 = jnp.asarray(_I8)

    st_e = _stats1(e_p)
    Bis = jnp.asarray(BIS)
    q = qs[0]
    xb, xr = pl.pallas_call(
        _node_prep_body,
        out_shape=[jax.ShapeDtypeStruct((N, D), F32)] * 2)(
            x, q['bn_g'], q['bn_b'], q['root'], q['cb'])

    for i in range(NUM_MP):
        q = qs[i]
        last = i == NUM_MP - 1

        h1_p, st_h1 = _stage_lin(e_p, st_e, P, *q['em0'])
        h2_p, st_h2 = _stage_lin(h1_p, st_h1, P, *q['em1'])

        xj_p = _sc_gather1(xb, rowi).reshape(ROWS, 128)
        msg_p = _msg(xj_p, h2_p, st_h2, P, q['m_g'], q['m_b'], q['W2s'],
                     q['b2s'], Bis)
        agg2 = _sc_scatter(msg_p.reshape(E, D), coli, zeros)

        if not last:
            qn = qs[i + 1]
            xn, xb, xr = pl.pallas_call(
                _node_update_mid_body,
                out_shape=[jax.ShapeDtypeStruct((N, D), F32)] * 3)(
                    agg2, xr, qn['bn_g'], qn['bn_b'], qn['root'], qn['cb'])
        else:
            xn, x_pred = pl.pallas_call(
                _node_update_last_body,
                out_shape=[jax.ShapeDtypeStruct((N, D), F32),
                           jax.ShapeDtypeStruct((N, 2), F32)])(
                    agg2, xr, params['node_pred_w'],
                    params['node_pred_b'].reshape(1, 2))

        xs16, xd16 = _sc_gather2(xn, rowi, coli)
        xs_p = xs16.reshape(ROWS, 128)
        xd_p = xd16.reshape(ROWS, 128)
        st_xs, st_xd = _stats2(xs_p, xd_p)
        g1_p, st_g1 = _stage_el1(xs_p, xd_p, e_p, st_xs, st_xd, st_e, P,
                                 q['el1'])
        g2_p, st_g2 = _stage_lin(g1_p, st_g1, P, *q['el2'])
        if not last:
            e_p, st_e = _stage_lin(g2_p, st_g2, P, *q['el3'], elu3=True)
        else:
            PWp = jnp.kron(eye8, params['edge_pred_w'])
            pb = jnp.tile(params['edge_pred_b'], PACK).reshape(1, 16)
            e_p, ep16 = _stage_el3_last(g2_p, st_g2, P, *q['el3'], PWp, pb)

    e_pred = ep16.reshape(E, 2)
    e3 = e_p.reshape(E, D)
    return (x_pred, e_pred, xn, e3)
